# Initial kernel scaffold; baseline (speedup 1.0000x reference)
import jax
import jax.numpy as jnp
from jax.experimental import pallas as pl


def kernel(x_n, edge_index, edge_attr, W1, b1, W2, b2, W3, b3, Wp, bp):
    raise NotImplementedError("write your pallas kernel here")


# speedup vs baseline: 12.0709x; 12.0709x over previous
import functools

import jax
import jax.numpy as jnp
from jax import lax
from jax.experimental import pallas as pl
from jax.experimental.pallas import tpu as pltpu
from jax.experimental.pallas import tpu_sc as plsc

N = 10000
N_PAD = 10240
E = 640000
NC, NS, L = 2, 16, 16
NW = NC * NS
CHUNK = 128
EPW = 20096
E_PAD = EPW * NW
CHUNKS_PER_W = EPW // CHUNK
ROWS_PER_TILE = N_PAD // NS

_mesh = plsc.VectorSubcoreMesh(core_axis_name="c", subcore_axis_name="s")
_sc_params = pltpu.CompilerParams(needs_layout_passes=False,
                                  use_tc_tiling_on_sc=False)


def _worker_base(cid, sid):
    return (cid * NS + sid) * EPW


@functools.partial(
    pl.kernel,
    out_type=jax.ShapeDtypeStruct((NC, N_PAD), jnp.float32),
    mesh=_mesh,
    scratch_types=[
        pltpu.VMEM((CHUNK,), jnp.int32),
        pltpu.VMEM((CHUNK,), jnp.float32),
        pltpu.VMEM((ROWS_PER_TILE,), jnp.float32),
        pltpu.VMEM_SHARED((N_PAD,), jnp.float32),
    ],
    compiler_params=_sc_params,
)
def _sc_deg(dst_hbm, ew_hbm, out_hbm, dst_v, ew_v, zeros_v, acc_sh):
    cid = lax.axis_index("c")
    sid = lax.axis_index("s")
    zz = jnp.zeros((L,), jnp.float32)

    def zfill(i, c):
        zeros_v[pl.ds(i * L, L)] = zz
        return c

    lax.fori_loop(0, ROWS_PER_TILE // L, zfill, 0)
    pltpu.sync_copy(zeros_v, acc_sh.at[pl.ds(sid * ROWS_PER_TILE, ROWS_PER_TILE)])
    plsc.subcore_barrier()

    base = _worker_base(cid, sid)

    def body(i, c):
        off = base + i * CHUNK
        pltpu.sync_copy(dst_hbm.at[pl.ds(off, CHUNK)], dst_v)
        pltpu.sync_copy(ew_hbm.at[pl.ds(off, CHUNK)], ew_v)
        pltpu.sync_copy(ew_v, acc_sh.at[dst_v], add=True)
        return c

    lax.fori_loop(0, CHUNKS_PER_W, body, 0)
    plsc.subcore_barrier()
    pltpu.sync_copy(
        acc_sh.at[pl.ds(sid * ROWS_PER_TILE, ROWS_PER_TILE)],
        out_hbm.at[cid, pl.ds(sid * ROWS_PER_TILE, ROWS_PER_TILE)],
    )


@functools.partial(
    pl.kernel,
    out_type=(
        jax.ShapeDtypeStruct((E_PAD,), jnp.float32),
        jax.ShapeDtypeStruct((NC, N_PAD), jnp.float32),
    ),
    mesh=_mesh,
    scratch_types=[
        pltpu.VMEM((N_PAD,), jnp.float32),
        pltpu.VMEM((N_PAD,), jnp.float32),
        pltpu.VMEM((CHUNK,), jnp.int32),
        pltpu.VMEM((CHUNK,), jnp.int32),
        pltpu.VMEM((CHUNK,), jnp.float32),
        pltpu.VMEM((CHUNK,), jnp.float32),
        pltpu.VMEM((CHUNK,), jnp.float32),
        pltpu.VMEM((ROWS_PER_TILE,), jnp.float32),
        pltpu.VMEM_SHARED((N_PAD,), jnp.float32),
    ],
    compiler_params=_sc_params,
)
def _sc_norm_s0(src_hbm, dst_hbm, ew_hbm, dinv_hbm, x_hbm, norm_hbm, s0_hbm,
                dinv_v, x_v, src_v, dst_v, ew_v, nm_v, msg_v, zeros_v, acc_sh):
    cid = lax.axis_index("c")
    sid = lax.axis_index("s")
    zz = jnp.zeros((L,), jnp.float32)

    def zfill(i, c):
        zeros_v[pl.ds(i * L, L)] = zz
        return c

    lax.fori_loop(0, ROWS_PER_TILE // L, zfill, 0)
    pltpu.sync_copy(zeros_v, acc_sh.at[pl.ds(sid * ROWS_PER_TILE, ROWS_PER_TILE)])
    pltpu.sync_copy(dinv_hbm, dinv_v)
    pltpu.sync_copy(x_hbm, x_v)
    plsc.subcore_barrier()

    base = _worker_base(cid, sid)

    def body(i, c):
        off = base + i * CHUNK
        pltpu.sync_copy(src_hbm.at[pl.ds(off, CHUNK)], src_v)
        pltpu.sync_copy(dst_hbm.at[pl.ds(off, CHUNK)], dst_v)
        pltpu.sync_copy(ew_hbm.at[pl.ds(off, CHUNK)], ew_v)
        for g in range(CHUNK // L):
            s16 = src_v[pl.ds(g * L, L)]
            d16 = dst_v[pl.ds(g * L, L)]
            e16 = ew_v[pl.ds(g * L, L)]
            a = plsc.load_gather(dinv_v, [s16])
            b = plsc.load_gather(dinv_v, [d16])
            xv = plsc.load_gather(x_v, [s16])
            nm = a * e16 * b
            nm_v[pl.ds(g * L, L)] = nm
            msg_v[pl.ds(g * L, L)] = nm * xv
        pltpu.sync_copy(nm_v, norm_hbm.at[pl.ds(off, CHUNK)])
        pltpu.sync_copy(msg_v, acc_sh.at[dst_v], add=True)
        return c

    lax.fori_loop(0, CHUNKS_PER_W, body, 0)
    plsc.subcore_barrier()
    pltpu.sync_copy(
        acc_sh.at[pl.ds(sid * ROWS_PER_TILE, ROWS_PER_TILE)],
        s0_hbm.at[cid, pl.ds(sid * ROWS_PER_TILE, ROWS_PER_TILE)],
    )


def _make_sc_agg(F):
    @functools.partial(
        pl.kernel,
        out_type=jax.ShapeDtypeStruct((NC, N_PAD, F), jnp.float32),
        mesh=_mesh,
        scratch_types=[
            pltpu.VMEM((CHUNK,), jnp.int32),
            pltpu.VMEM((CHUNK,), jnp.int32),
            pltpu.VMEM((CHUNK + L,), jnp.float32),
            pltpu.VMEM((CHUNK, F), jnp.float32),
            pltpu.VMEM((L, F), jnp.float32),
            pltpu.VMEM_SHARED((N_PAD, F), jnp.float32),
            pltpu.SemaphoreType.DMA,
        ],
        compiler_params=_sc_params,
    )
    def _sc_agg(src_hbm, dst_hbm, nm_hbm, h_hbm, out_hbm,
                src_v, dst_v, nm_v, rows_v, zeros_v, acc_sh, sem):
        cid = lax.axis_index("c")
        sid = lax.axis_index("s")
        zz = jnp.zeros((L,), jnp.float32)
        for r in range(L):
            for t in range(F // L):
                zeros_v[r, pl.ds(t * L, L)] = zz

        def zcopy(k, c):
            pltpu.sync_copy(
                zeros_v, acc_sh.at[pl.ds(sid * ROWS_PER_TILE + k * L, L), :])
            return c

        lax.fori_loop(0, ROWS_PER_TILE // L, zcopy, 0)
        plsc.subcore_barrier()

        base = _worker_base(cid, sid)

        def body(i, c):
            off = base + i * CHUNK
            pltpu.sync_copy(src_hbm.at[pl.ds(off, CHUNK)], src_v)
            pltpu.sync_copy(dst_hbm.at[pl.ds(off, CHUNK)], dst_v)
            pltpu.sync_copy(nm_hbm.at[pl.ds(off, CHUNK)],
                            nm_v.at[pl.ds(L, CHUNK)])
            pltpu.async_copy(h_hbm.at[src_v], rows_v, sem).wait()
            for r in range(CHUNK):
                idx = jnp.full((L,), L + r, jnp.int32)
                nv = plsc.load_gather(nm_v, [idx])
                for t in range(F // L):
                    rows_v[r, pl.ds(t * L, L)] = rows_v[r, pl.ds(t * L, L)] * nv
            pltpu.sync_copy(rows_v, acc_sh.at[dst_v], add=True)
            return c

        lax.fori_loop(0, CHUNKS_PER_W, body, 0)
        plsc.subcore_barrier()

        def dump(k, c):
            row0 = sid * ROWS_PER_TILE + k * L
            pltpu.sync_copy(acc_sh.at[pl.ds(row0, L), :],
                            out_hbm.at[cid, pl.ds(row0, L), :])
            return c

        lax.fori_loop(0, ROWS_PER_TILE // L, dump, 0)

    return _sc_agg


_sc_agg64 = _make_sc_agg(64)
_sc_agg128 = _make_sc_agg(128)


def _tc_dinv_body(degp_ref, dinv_ref, dinv2_ref):
    deg = degp_ref[0] + degp_ref[1] + 1.0
    di = lax.rsqrt(deg)
    dinv_ref[...] = di
    dinv2_ref[...] = di * di


_tc_dinv = pl.pallas_call(
    _tc_dinv_body,
    out_shape=(
        jax.ShapeDtypeStruct((N_PAD // 128, 128), jnp.float32),
        jax.ShapeDtypeStruct((N_PAD // 128, 128), jnp.float32),
    ),
)


def _tc_h1_body(s0p_ref, x_ref, dinv2_ref, w1_ref, b1_ref, h1_ref):
    s0 = s0p_ref[0] + s0p_ref[1] + dinv2_ref[...] * x_ref[...]
    h1_ref[...] = jnp.maximum(s0 * w1_ref[...] + b1_ref[...], 0.0)


_tc_h1 = pl.pallas_call(
    _tc_h1_body,
    out_shape=jax.ShapeDtypeStruct((N_PAD, 64), jnp.float32),
)


def _tc_layer_body(gp_ref, h_ref, dinv2_ref, w_ref, b_ref, out_ref):
    g = gp_ref[0] + gp_ref[1] + dinv2_ref[...] * h_ref[...]
    out_ref[...] = jnp.maximum(
        jnp.dot(g, w_ref[...], preferred_element_type=jnp.float32)
        + b_ref[...], 0.0)


def _make_tc_layer(f_out):
    return pl.pallas_call(
        _tc_layer_body,
        out_shape=jax.ShapeDtypeStruct((N_PAD, f_out), jnp.float32),
    )


_tc_layer2 = _make_tc_layer(128)


def _tc_out_body(gp_ref, h_ref, dinv2_ref, w3_ref, b3_ref, wp_ref, bp_ref,
                 out_ref):
    g = gp_ref[0] + gp_ref[1] + dinv2_ref[...] * h_ref[...]
    h3 = jnp.maximum(
        jnp.dot(g, w3_ref[...], preferred_element_type=jnp.float32)
        + b3_ref[...], 0.0)
    out_ref[...] = (
        jnp.dot(h3, wp_ref[...], preferred_element_type=jnp.float32)
        + bp_ref[...])


_tc_out = pl.pallas_call(
    _tc_out_body,
    out_shape=jax.ShapeDtypeStruct((N_PAD, 1), jnp.float32),
)


def kernel(x_n, edge_index, edge_attr, W1, b1, W2, b2, W3, b3, Wp, bp):
    pad = E_PAD - E
    src = jnp.concatenate([edge_index[0], jnp.zeros((pad,), jnp.int32)])
    dst = jnp.concatenate([edge_index[1], jnp.zeros((pad,), jnp.int32)])
    ew = jnp.concatenate([edge_attr, jnp.zeros((pad,), jnp.float32)])
    x_pad = jnp.pad(x_n[:, 0], (0, N_PAD - N))

    degp = _sc_deg(dst, ew)
    dinv, dinv2 = _tc_dinv(degp.reshape(NC, N_PAD // 128, 128))
    dinv = dinv.reshape(N_PAD)
    dinv2 = dinv2.reshape(N_PAD, 1)

    norm, s0p = _sc_norm_s0(src, dst, ew, dinv, x_pad)

    h1 = _tc_h1(s0p.reshape(NC, N_PAD, 1), x_pad.reshape(N_PAD, 1), dinv2,
                W1, b1.reshape(1, 64))
    g1p = _sc_agg64(src, dst, norm, h1)
    h2 = _tc_layer2(g1p, h1, dinv2, W2, b2.reshape(1, 128))
    g2p = _sc_agg128(src, dst, norm, h2)
    out = _tc_out(g2p, h2, dinv2, W3, b3.reshape(1, 256), Wp,
                  bp.reshape(1, 1))
    return out[:N]


# speedup vs baseline: 19.3748x; 1.6051x over previous
import functools

import jax
import jax.numpy as jnp
from jax import lax
from jax.experimental import pallas as pl
from jax.experimental.pallas import tpu as pltpu
from jax.experimental.pallas import tpu_sc as plsc

N = 10000
N_PAD = 10240
E = 640000
NC, NS, L = 2, 16, 16
NW = NC * NS
CHUNK = 128
CHUNKS_PER_W = 158
EPW = CHUNKS_PER_W * CHUNK
E_PAD = EPW * NW
ROWS_PER_TILE = N_PAD // NS

_mesh = plsc.VectorSubcoreMesh(core_axis_name="c", subcore_axis_name="s")
_sc_params = pltpu.CompilerParams(needs_layout_passes=False,
                                  use_tc_tiling_on_sc=False)


@functools.partial(
    pl.kernel,
    out_type=jax.ShapeDtypeStruct((NC, N_PAD), jnp.float32),
    mesh=_mesh,
    scratch_types=[
        pltpu.VMEM((CHUNKS_PER_W, CHUNK), jnp.int32),
        pltpu.VMEM((EPW,), jnp.float32),
        pltpu.VMEM((ROWS_PER_TILE,), jnp.float32),
        pltpu.VMEM_SHARED((N_PAD,), jnp.float32),
    ],
    compiler_params=_sc_params,
)
def _sc_deg(dst_hbm, ew_hbm, out_hbm, dst_v, ew_v, zeros_v, acc_sh):
    cid = lax.axis_index("c")
    sid = lax.axis_index("s")
    zz = jnp.zeros((L,), jnp.float32)

    def zfill(i, c):
        zeros_v[pl.ds(i * L, L)] = zz
        return c

    lax.fori_loop(0, ROWS_PER_TILE // L, zfill, 0)
    pltpu.sync_copy(zeros_v, acc_sh.at[pl.ds(sid * ROWS_PER_TILE, ROWS_PER_TILE)])

    wid = cid * NS + sid
    pltpu.sync_copy(
        dst_hbm.at[pl.ds(wid * CHUNKS_PER_W, CHUNKS_PER_W), :], dst_v)
    pltpu.sync_copy(ew_hbm.at[pl.ds(wid * EPW, EPW)], ew_v)
    plsc.subcore_barrier()

    def body(i, c):
        pltpu.sync_copy(ew_v.at[pl.ds(i * CHUNK, CHUNK)],
                        acc_sh.at[dst_v.at[i]], add=True)
        return c

    lax.fori_loop(0, CHUNKS_PER_W, body, 0)
    plsc.subcore_barrier()
    pltpu.sync_copy(
        acc_sh.at[pl.ds(sid * ROWS_PER_TILE, ROWS_PER_TILE)],
        out_hbm.at[cid, pl.ds(sid * ROWS_PER_TILE, ROWS_PER_TILE)],
    )


@functools.partial(
    pl.kernel,
    out_type=(
        jax.ShapeDtypeStruct((E_PAD,), jnp.float32),
        jax.ShapeDtypeStruct((NC, N_PAD), jnp.float32),
    ),
    mesh=_mesh,
    scratch_types=[
        pltpu.VMEM((N_PAD,), jnp.float32),
        pltpu.VMEM((N_PAD,), jnp.float32),
        pltpu.VMEM((EPW,), jnp.int32),
        pltpu.VMEM((CHUNKS_PER_W, CHUNK), jnp.int32),
        pltpu.VMEM((EPW,), jnp.float32),
        pltpu.VMEM((EPW,), jnp.float32),
        pltpu.VMEM((EPW,), jnp.float32),
        pltpu.VMEM((ROWS_PER_TILE,), jnp.float32),
        pltpu.VMEM_SHARED((N_PAD,), jnp.float32),
    ],
    compiler_params=_sc_params,
)
def _sc_norm_s0(src_hbm, dst_hbm, ew_hbm, dinv_hbm, x_hbm, norm_hbm, s0_hbm,
                dinv_v, x_v, src_v, dst_v, ew_v, nm_v, msg_v, zeros_v, acc_sh):
    cid = lax.axis_index("c")
    sid = lax.axis_index("s")
    zz = jnp.zeros((L,), jnp.float32)

    def zfill(i, c):
        zeros_v[pl.ds(i * L, L)] = zz
        return c

    lax.fori_loop(0, ROWS_PER_TILE // L, zfill, 0)
    pltpu.sync_copy(zeros_v, acc_sh.at[pl.ds(sid * ROWS_PER_TILE, ROWS_PER_TILE)])

    wid = cid * NS + sid
    pltpu.sync_copy(src_hbm.at[pl.ds(wid * EPW, EPW)], src_v)
    pltpu.sync_copy(
        dst_hbm.at[pl.ds(wid * CHUNKS_PER_W, CHUNKS_PER_W), :], dst_v)
    pltpu.sync_copy(ew_hbm.at[pl.ds(wid * EPW, EPW)], ew_v)
    pltpu.sync_copy(dinv_hbm, dinv_v)
    pltpu.sync_copy(x_hbm, x_v)
    plsc.subcore_barrier()

    def grp(g, c):
        s16 = src_v[pl.ds(g * L, L)]
        e16 = ew_v[pl.ds(g * L, L)]
        a = plsc.load_gather(dinv_v, [s16])
        xv = plsc.load_gather(x_v, [s16])
        nm = a * e16
        nm_v[pl.ds(g * L, L)] = nm
        msg_v[pl.ds(g * L, L)] = nm * xv
        return c

    lax.fori_loop(0, EPW // L, grp, 0)

    def body(i, c):
        pltpu.sync_copy(msg_v.at[pl.ds(i * CHUNK, CHUNK)],
                        acc_sh.at[dst_v.at[i]], add=True)
        return c

    lax.fori_loop(0, CHUNKS_PER_W, body, 0)
    pltpu.sync_copy(nm_v, norm_hbm.at[pl.ds(wid * EPW, EPW)])
    plsc.subcore_barrier()
    pltpu.sync_copy(
        acc_sh.at[pl.ds(sid * ROWS_PER_TILE, ROWS_PER_TILE)],
        s0_hbm.at[cid, pl.ds(sid * ROWS_PER_TILE, ROWS_PER_TILE)],
    )


FH = 64


def _make_sc_agg(n_parts):
    ZROWS = 32

    @functools.partial(
        pl.kernel,
        out_type=jax.ShapeDtypeStruct((NC, n_parts, N_PAD, FH), jnp.float32),
        mesh=_mesh,
        scratch_types=[
            pltpu.VMEM((EPW,), jnp.int32),
            pltpu.VMEM((CHUNKS_PER_W, CHUNK), jnp.int32),
            pltpu.VMEM((L + EPW,), jnp.float32),
            pltpu.VMEM((CHUNK, FH), jnp.float32),
            pltpu.VMEM((CHUNK, FH), jnp.float32),
            pltpu.VMEM((ZROWS, FH), jnp.float32),
            pltpu.VMEM_SHARED((N_PAD, FH), jnp.float32),
            pltpu.SemaphoreType.DMA,
            pltpu.SemaphoreType.DMA,
        ],
        compiler_params=_sc_params,
    )
    def _sc_agg(src_hbm, dst_hbm, nm_hbm, *rest):
        h_parts = rest[:n_parts]
        out_hbm = rest[n_parts]
        (src_v, dst_v, nm_v, rows_a, rows_b, zeros_v, acc_sh,
         sem_a, sem_b) = rest[n_parts + 1:]
        cid = lax.axis_index("c")
        sid = lax.axis_index("s")
        zz = jnp.zeros((L,), jnp.float32)
        for r in range(ZROWS):
            for t in range(FH // L):
                zeros_v[r, pl.ds(t * L, L)] = zz

        wid = cid * NS + sid
        pltpu.sync_copy(src_hbm.at[pl.ds(wid * EPW, EPW)], src_v)
        pltpu.sync_copy(
            dst_hbm.at[pl.ds(wid * CHUNKS_PER_W, CHUNKS_PER_W), :], dst_v)
        pltpu.sync_copy(nm_hbm.at[pl.ds(wid * EPW, EPW)],
                        nm_v.at[pl.ds(L, EPW)])

        def scale(i, buf):
            base = i * CHUNK + L
            for r in range(CHUNK):
                nv = plsc.load_gather(nm_v, [jnp.full((L,), r, jnp.int32) + base])
                for t in range(FH // L):
                    buf[r, pl.ds(t * L, L)] = buf[r, pl.ds(t * L, L)] * nv

        for part, h_hbm in enumerate(h_parts):
            def zcopy(k, c):
                pltpu.sync_copy(
                    zeros_v,
                    acc_sh.at[pl.ds(sid * ROWS_PER_TILE + k * ZROWS, ZROWS), :])
                return c

            lax.fori_loop(0, ROWS_PER_TILE // ZROWS, zcopy, 0)
            plsc.subcore_barrier()

            def gather(i, buf, sem):
                pltpu.async_copy(h_hbm.at[src_v.at[pl.ds(i * CHUNK, CHUNK)]],
                                 buf, sem)

            def drain(buf, sem):
                pltpu.make_async_copy(h_hbm.at[pl.ds(0, CHUNK), :], buf,
                                      sem).wait()

            gather(0, rows_a, sem_a)

            def body(k, c):
                i0 = 2 * k
                gather(i0 + 1, rows_b, sem_b)
                drain(rows_a, sem_a)
                scale(i0, rows_a)
                pltpu.sync_copy(rows_a, acc_sh.at[dst_v.at[i0]], add=True)

                @pl.when(i0 + 2 < CHUNKS_PER_W)
                def _():
                    gather(i0 + 2, rows_a, sem_a)

                drain(rows_b, sem_b)
                scale(i0 + 1, rows_b)
                pltpu.sync_copy(rows_b, acc_sh.at[dst_v.at[i0 + 1]], add=True)
                return c

            lax.fori_loop(0, CHUNKS_PER_W // 2, body, 0)
            plsc.subcore_barrier()
            pltpu.sync_copy(
                acc_sh.at[pl.ds(sid * ROWS_PER_TILE, ROWS_PER_TILE), :],
                out_hbm.at[cid, part,
                           pl.ds(sid * ROWS_PER_TILE, ROWS_PER_TILE), :])
            plsc.subcore_barrier()

    return _sc_agg


_sc_agg64 = _make_sc_agg(1)
_sc_agg128 = _make_sc_agg(2)


def _tc_dinv_body(degp_ref, dinv_ref, dinv2_ref):
    deg = degp_ref[0] + degp_ref[1] + 1.0
    di = lax.rsqrt(deg)
    dinv_ref[...] = di
    dinv2_ref[...] = di * di


_tc_dinv = pl.pallas_call(
    _tc_dinv_body,
    out_shape=(
        jax.ShapeDtypeStruct((N_PAD // 128, 128), jnp.float32),
        jax.ShapeDtypeStruct((N_PAD // 128, 128), jnp.float32),
    ),
)


def _tc_h1_body(s0p_ref, x_ref, dinv_ref, dinv2_ref, w1_ref, b1_ref, h1_ref):
    s0 = (dinv_ref[...] * (s0p_ref[0] + s0p_ref[1])
          + dinv2_ref[...] * x_ref[...])
    h1_ref[...] = jnp.maximum(s0 * w1_ref[...] + b1_ref[...], 0.0)


_tc_h1 = pl.pallas_call(
    _tc_h1_body,
    out_shape=jax.ShapeDtypeStruct((N_PAD, 64), jnp.float32),
)


def _tc_layer_body(gp_ref, h_ref, dinv_ref, dinv2_ref, w_ref, b_ref,
                   lo_ref, hi_ref):
    g = (dinv_ref[...] * (gp_ref[0] + gp_ref[1])
         + dinv2_ref[...] * h_ref[...])
    h2 = jnp.maximum(
        jnp.dot(g, w_ref[...], preferred_element_type=jnp.float32)
        + b_ref[...], 0.0)
    lo_ref[...] = h2[:, :FH]
    hi_ref[...] = h2[:, FH:]


_BR = 2048

_tc_layer2 = pl.pallas_call(
    _tc_layer_body,
    grid=(N_PAD // _BR,),
    in_specs=[
        pl.BlockSpec((NC, _BR, FH), lambda i: (0, i, 0)),
        pl.BlockSpec((_BR, FH), lambda i: (i, 0)),
        pl.BlockSpec((_BR, 1), lambda i: (i, 0)),
        pl.BlockSpec((_BR, 1), lambda i: (i, 0)),
        pl.BlockSpec((64, 128), lambda i: (0, 0)),
        pl.BlockSpec((1, 128), lambda i: (0, 0)),
    ],
    out_specs=(
        pl.BlockSpec((_BR, FH), lambda i: (i, 0)),
        pl.BlockSpec((_BR, FH), lambda i: (i, 0)),
    ),
    out_shape=(
        jax.ShapeDtypeStruct((N_PAD, FH), jnp.float32),
        jax.ShapeDtypeStruct((N_PAD, FH), jnp.float32),
    ),
)


def _tc_out_body(gp_ref, hlo_ref, hhi_ref, dinv_ref, dinv2_ref, w3_ref,
                 b3_ref, wp_ref, bp_ref, out_ref):
    gs = jnp.concatenate([gp_ref[0, 0] + gp_ref[1, 0],
                          gp_ref[0, 1] + gp_ref[1, 1]], axis=1)
    h2 = jnp.concatenate([hlo_ref[...], hhi_ref[...]], axis=1)
    g = dinv_ref[...] * gs + dinv2_ref[...] * h2
    h3 = jnp.maximum(
        jnp.dot(g, w3_ref[...], preferred_element_type=jnp.float32)
        + b3_ref[...], 0.0)
    out_ref[...] = (
        jnp.dot(h3, wp_ref[...], preferred_element_type=jnp.float32)
        + bp_ref[...])


_tc_out = pl.pallas_call(
    _tc_out_body,
    grid=(N_PAD // _BR,),
    in_specs=[
        pl.BlockSpec((NC, 2, _BR, FH), lambda i: (0, 0, i, 0)),
        pl.BlockSpec((_BR, FH), lambda i: (i, 0)),
        pl.BlockSpec((_BR, FH), lambda i: (i, 0)),
        pl.BlockSpec((_BR, 1), lambda i: (i, 0)),
        pl.BlockSpec((_BR, 1), lambda i: (i, 0)),
        pl.BlockSpec((128, 256), lambda i: (0, 0)),
        pl.BlockSpec((1, 256), lambda i: (0, 0)),
        pl.BlockSpec((256, 1), lambda i: (0, 0)),
        pl.BlockSpec((1, 1), lambda i: (0, 0)),
    ],
    out_specs=pl.BlockSpec((_BR, 1), lambda i: (i, 0)),
    out_shape=jax.ShapeDtypeStruct((N_PAD, 1), jnp.float32),
)


def kernel(x_n, edge_index, edge_attr, W1, b1, W2, b2, W3, b3, Wp, bp):
    pad = E_PAD - E
    src = jnp.concatenate([edge_index[0], jnp.zeros((pad,), jnp.int32)])
    dst = jnp.concatenate([edge_index[1], jnp.zeros((pad,), jnp.int32)])
    dst = dst.reshape(NW * CHUNKS_PER_W, CHUNK)
    ew = jnp.concatenate([edge_attr, jnp.zeros((pad,), jnp.float32)])
    x_pad = jnp.pad(x_n[:, 0], (0, N_PAD - N))

    degp = _sc_deg(dst, ew)
    dinv, dinv2 = _tc_dinv(degp.reshape(NC, N_PAD // 128, 128))
    dinv_flat = dinv.reshape(N_PAD)
    dinv_col = dinv.reshape(N_PAD, 1)
    dinv2_col = dinv2.reshape(N_PAD, 1)

    norm, s0p = _sc_norm_s0(src, dst, ew, dinv_flat, x_pad)

    h1 = _tc_h1(s0p.reshape(NC, N_PAD, 1), x_pad.reshape(N_PAD, 1), dinv_col,
                dinv2_col, W1, b1.reshape(1, 64))
    g1p = _sc_agg64(src, dst, norm, h1)
    h2_lo, h2_hi = _tc_layer2(g1p.reshape(NC, N_PAD, FH), h1, dinv_col,
                              dinv2_col, W2, b2.reshape(1, 128))
    g2p = _sc_agg128(src, dst, norm, h2_lo, h2_hi)
    out = _tc_out(g2p, h2_lo, h2_hi, dinv_col, dinv2_col, W3,
                  b3.reshape(1, 256), Wp, bp.reshape(1, 1))
    return out[:N]
